# baseline (speedup 1.0000x reference)
import functools

import jax
import jax.numpy as jnp
from jax import lax
from jax.experimental import pallas as pl
from jax.experimental.pallas import tpu as pltpu
from jax.experimental.pallas import tpu_sc as plsc

_NW = 32
_NBUF = 2
_K = 4


def _make_chunk(Bc, S, V, SL):
    b_per_w = Bc // _NW
    mesh = plsc.VectorSubcoreMesh(core_axis_name="c", subcore_axis_name="s")

    @functools.partial(
        pl.kernel,
        mesh=mesh,
        out_type=jax.ShapeDtypeStruct((Bc, S, SL, 128), jnp.float32),
        scratch_types=[
            pltpu.VMEM((b_per_w, S), jnp.int32),
            pltpu.VMEM((S, SL, 128), jnp.float32),
            pltpu.VMEM((S, SL, 128), jnp.float32),
            pltpu.SemaphoreType.DMA,
            pltpu.SemaphoreType.DMA,
            pltpu.SemaphoreType.DMA,
            pltpu.SemaphoreType.DMA,
        ],
    )
    def _gather(table_hbm, idx_hbm, out_hbm, idx_v, buf0, buf1,
                gsem0, gsem1, ssem0, ssem1):
        wid = lax.axis_index("s") * 2 + lax.axis_index("c")
        base = wid * b_per_w
        pltpu.sync_copy(idx_hbm.at[wid], idx_v)

        bufs = (buf0, buf1)
        gsems = (gsem0, gsem1)
        ssems = (ssem0, ssem1)

        def start_gather(bl, p):
            pltpu.async_copy(
                table_hbm.at[idx_v.at[bl]], bufs[p], gsems[p])

        def start_scatter(bl, p):
            pltpu.async_copy(bufs[p], out_hbm.at[base + bl], ssems[p])

        def wait_gather(p):
            pltpu.make_async_copy(
                table_hbm.at[idx_v.at[0]], bufs[p], gsems[p]).wait()

        def wait_scatter(p):
            pltpu.make_async_copy(
                bufs[p], out_hbm.at[base], ssems[p]).wait()

        for p in range(_NBUF):
            start_gather(p, p)

        def body(g, carry):
            bl = g * _NBUF
            for p in range(_NBUF):
                blp = bl + p

                @pl.when(blp < b_per_w)
                def _():
                    wait_gather(p)
                    start_scatter(blp, p)
                    wait_scatter(p)

                    @pl.when(blp + _NBUF < b_per_w)
                    def _():
                        start_gather(blp + _NBUF, p)

            return carry

        lax.fori_loop(0, (b_per_w + _NBUF - 1) // _NBUF, body, 0)

    return _gather


def kernel(tokens, lookup_table):
    B, S = tokens.shape
    V, D = lookup_table.shape
    Dp = 1024
    SL = Dp // 128
    Bc = B // _K
    b_per_w = Bc // _NW
    table3 = jnp.pad(lookup_table, ((0, 0), (0, Dp - D))).reshape(V, SL, 128)
    idx4 = tokens.astype(jnp.int32).reshape(_K, _NW, b_per_w, S)
    chunk = _make_chunk(Bc, S, V, SL)
    outs = [
        chunk(table3, idx4[k]).reshape(Bc, S, Dp)[:, :, :D]
        for k in range(_K)
    ]
    return jnp.concatenate(outs, axis=0)


# speedup vs baseline: 1.1191x; 1.1191x over previous
import functools

import jax
import jax.numpy as jnp
from jax import lax
from jax.experimental import pallas as pl
from jax.experimental.pallas import tpu as pltpu
from jax.experimental.pallas import tpu_sc as plsc

_NW = 32
_NBUF = 2


def kernel(tokens, lookup_table):
    B, S = tokens.shape
    V, D = lookup_table.shape
    Dp = 1024
    SL = Dp // 128
    b_per_w = B // _NW
    idx3 = tokens.astype(jnp.int32).reshape(_NW, b_per_w, S)
    table3 = jnp.pad(lookup_table, ((0, 0), (0, Dp - D))).reshape(V, SL, 128)

    mesh = plsc.VectorSubcoreMesh(core_axis_name="c", subcore_axis_name="s")

    @functools.partial(
        pl.kernel,
        mesh=mesh,
        out_type=jax.ShapeDtypeStruct((B, S, SL, 128), jnp.float32),
        scratch_types=[
            pltpu.VMEM((b_per_w, S), jnp.int32),
            pltpu.VMEM((S, SL, 128), jnp.float32),
            pltpu.VMEM((S, SL, 128), jnp.float32),
            pltpu.SemaphoreType.DMA,
            pltpu.SemaphoreType.DMA,
            pltpu.SemaphoreType.DMA,
            pltpu.SemaphoreType.DMA,
        ],
    )
    def _gather(table_hbm, idx_hbm, out_hbm, idx_v, buf0, buf1,
                gsem0, gsem1, ssem0, ssem1):
        wid = lax.axis_index("s") * 2 + lax.axis_index("c")
        base = wid * b_per_w
        pltpu.sync_copy(idx_hbm.at[wid], idx_v)

        bufs = (buf0, buf1)
        gsems = (gsem0, gsem1)
        ssems = (ssem0, ssem1)

        def start_gather(bl, p):
            pltpu.async_copy(
                table_hbm.at[idx_v.at[bl]], bufs[p], gsems[p])

        def start_scatter(bl, p):
            pltpu.async_copy(bufs[p], out_hbm.at[base + bl], ssems[p])

        def wait_gather(p):
            pltpu.make_async_copy(
                table_hbm.at[idx_v.at[0]], bufs[p], gsems[p]).wait()

        def wait_scatter(p):
            pltpu.make_async_copy(
                bufs[p], out_hbm.at[base], ssems[p]).wait()

        for p in range(_NBUF):
            start_gather(p, p)

        def body(g, carry):
            bl = g * _NBUF
            for p in range(_NBUF):
                blp = bl + p

                @pl.when(blp < b_per_w)
                def _():
                    wait_gather(p)
                    start_scatter(blp, p)
                    wait_scatter(p)

                    @pl.when(blp + _NBUF < b_per_w)
                    def _():
                        start_gather(blp + _NBUF, p)

            return carry

        lax.fori_loop(0, (b_per_w + _NBUF - 1) // _NBUF, body, 0)

    out4 = _gather(table3, idx3)
    head = out4[:, :, :7, :].reshape(B, S, 7 * 128)
    tail = out4[:, :, 7:, :D - 7 * 128].reshape(B, S, D - 7 * 128)
    return jnp.concatenate([head, tail], axis=2)


# speedup vs baseline: 1.2314x; 1.1004x over previous
import functools

import jax
import jax.numpy as jnp
from jax import lax
from jax.experimental import pallas as pl
from jax.experimental.pallas import tpu as pltpu
from jax.experimental.pallas import tpu_sc as plsc

_NW = 32
_NBUF = 2


def kernel(tokens, lookup_table):
    B, S = tokens.shape
    V, D = lookup_table.shape
    Dp = 1024
    SL = Dp // 128
    b_per_w = B // _NW
    idx3 = tokens.astype(jnp.int32).reshape(_NW, b_per_w, S)
    table3 = jnp.pad(lookup_table, ((0, 0), (0, Dp - D))).reshape(V, SL, 128)

    mesh = plsc.VectorSubcoreMesh(core_axis_name="c", subcore_axis_name="s")

    @functools.partial(
        pl.kernel,
        mesh=mesh,
        out_type=jax.ShapeDtypeStruct((B, S, SL, 128), jnp.float32),
        scratch_types=[
            pltpu.VMEM((b_per_w, S), jnp.int32),
            pltpu.VMEM((S, SL, 128), jnp.float32),
            pltpu.VMEM((S, SL, 128), jnp.float32),
            pltpu.SemaphoreType.DMA,
            pltpu.SemaphoreType.DMA,
            pltpu.SemaphoreType.DMA,
            pltpu.SemaphoreType.DMA,
        ],
    )
    def _gather(table_hbm, idx_hbm, out_hbm, idx_v, buf0, buf1,
                gsem0, gsem1, ssem0, ssem1):
        wid = lax.axis_index("s") * 2 + lax.axis_index("c")
        base = wid * b_per_w
        pltpu.sync_copy(idx_hbm.at[wid], idx_v)

        bufs = (buf0, buf1)
        gsems = (gsem0, gsem1)
        ssems = (ssem0, ssem1)

        def start_gather(bl, p):
            pltpu.async_copy(
                table_hbm.at[idx_v.at[bl]], bufs[p], gsems[p])

        def start_scatter(bl, p):
            pltpu.async_copy(bufs[p], out_hbm.at[base + bl], ssems[p])

        def wait_gather(p):
            pltpu.make_async_copy(
                table_hbm.at[idx_v.at[0]], bufs[p], gsems[p]).wait()

        def wait_scatter(p):
            pltpu.make_async_copy(
                bufs[p], out_hbm.at[base], ssems[p]).wait()

        for p in range(_NBUF):
            start_gather(p, p)

        def body(g, carry):
            bl = g * _NBUF
            for p in range(_NBUF):
                blp = bl + p

                @pl.when(blp < b_per_w)
                def _():
                    wait_gather(p)
                    start_scatter(blp, p)
                    wait_scatter(p)

                    @pl.when(blp + _NBUF < b_per_w)
                    def _():
                        start_gather(blp + _NBUF, p)

            return carry

        lax.fori_loop(0, (b_per_w + _NBUF - 1) // _NBUF, body, 0)

    out4 = _gather(table3, idx3)
    return out4.reshape(B, S, Dp)[:, :, :D]
